# Initial kernel scaffold; baseline (speedup 1.0000x reference)
import jax
import jax.numpy as jnp
from jax import lax
from jax.experimental import pallas as pl
from jax.experimental.pallas import tpu as pltpu
from jax.experimental.pallas import tpu_sc as plsc

_GAMMA = 12.0
_N_EDGES = 320000
_D = 128
_NW = 32
_EPW = _N_EDGES // _NW
_CHUNK = 80
_NCHUNK = _EPW // _CHUNK
_G16 = _CHUNK // 16


def _sc_body(node_hbm, src_hbm, dst_hbm, rel_hbm, out_hbm,
             src_v, dst_v, head_v, tail_v, rel_v, score_v, sem):
    wid = lax.axis_index("s") * 2 + lax.axis_index("c")
    lanes = lax.iota(jnp.int32, 16)

    def chunk_body(i, carry):
        base = wid * _EPW + i * _CHUNK
        pltpu.sync_copy(src_hbm.at[pl.ds(base, _CHUNK)], src_v)
        pltpu.sync_copy(dst_hbm.at[pl.ds(base, _CHUNK)], dst_v)
        pltpu.async_copy(node_hbm.at[src_v], head_v, sem).wait()
        pltpu.async_copy(node_hbm.at[dst_v], tail_v, sem).wait()
        pltpu.sync_copy(rel_hbm.at[pl.ds(base, _CHUNK)], rel_v)

        def group_body(g, carry2):
            def edge_body(e, sv):
                row = g * 16 + e
                acc = jnp.zeros((16,), jnp.float32)
                for j in range(8):
                    h = head_v[row, pl.ds(j * 16, 16)]
                    r = rel_v[row, pl.ds(j * 16, 16)]
                    t = tail_v[row, pl.ds(j * 16, 16)]
                    acc = acc + jnp.abs(h + r - t)
                s = jnp.sum(acc)
                return jnp.where(lanes == e, _GAMMA - s, sv)

            sv = lax.fori_loop(0, 16, edge_body, jnp.zeros((16,), jnp.float32))
            score_v[pl.ds(g * 16, 16)] = sv
            return carry2

        lax.fori_loop(0, _G16, group_body, 0)
        pltpu.sync_copy(score_v, out_hbm.at[pl.ds(base, _CHUNK)])
        return carry

    lax.fori_loop(0, _NCHUNK, chunk_body, 0)


def kernel(node_emb, edge_index, rel_emb):
    src = edge_index[0].astype(jnp.int32)
    dst = edge_index[1].astype(jnp.int32)
    mesh = plsc.VectorSubcoreMesh(core_axis_name="c", subcore_axis_name="s")
    f = pl.kernel(
        _sc_body,
        out_type=jax.ShapeDtypeStruct((_N_EDGES,), jnp.float32),
        mesh=mesh,
        scratch_types=[
            pltpu.VMEM((_CHUNK,), jnp.int32),
            pltpu.VMEM((_CHUNK,), jnp.int32),
            pltpu.VMEM((_CHUNK, _D), jnp.float32),
            pltpu.VMEM((_CHUNK, _D), jnp.float32),
            pltpu.VMEM((_CHUNK, _D), jnp.float32),
            pltpu.VMEM((_CHUNK,), jnp.float32),
            pltpu.SemaphoreType.DMA,
        ],
    )
    return f(node_emb, src, dst, rel_emb)


# speedup vs baseline: 2.0968x; 2.0968x over previous
import jax
import jax.numpy as jnp
from jax import lax
from jax.experimental import pallas as pl
from jax.experimental.pallas import tpu as pltpu
from jax.experimental.pallas import tpu_sc as plsc

_GAMMA = 12.0
_N_EDGES = 320000
_D = 128
_NW = 32
_EPW = _N_EDGES // _NW
_CHUNK = 80
_NCHUNK = _EPW // _CHUNK
_G16 = _CHUNK // 16


def _sc_body(node_hbm, src_hbm, dst_hbm, rel_hbm, out_hbm,
             src_v, dst_v, head_v, tail_v, rel_v, score_v, acc_buf, sem):
    wid = lax.axis_index("s") * 2 + lax.axis_index("c")
    lanes = lax.iota(jnp.int32, 16)

    def chunk_body(i, carry):
        base = wid * _EPW + i * _CHUNK
        pltpu.sync_copy(src_hbm.at[pl.ds(base, _CHUNK)], src_v)
        pltpu.sync_copy(dst_hbm.at[pl.ds(base, _CHUNK)], dst_v)
        pltpu.async_copy(node_hbm.at[src_v], head_v, sem).wait()
        pltpu.async_copy(node_hbm.at[dst_v], tail_v, sem).wait()
        pltpu.sync_copy(rel_hbm.at[pl.ds(base, _CHUNK)], rel_v)

        def group_body(g, carry2):
            def edge_body(e, carry3):
                row = g * 16 + e
                acc = jnp.zeros((16,), jnp.float32)
                for j in range(8):
                    h = head_v[row, pl.ds(j * 16, 16)]
                    r = rel_v[row, pl.ds(j * 16, 16)]
                    t = tail_v[row, pl.ds(j * 16, 16)]
                    acc = acc + jnp.abs(h + r - t)
                acc_buf[e, pl.ds(0, 16)] = acc
                return carry3

            lax.fori_loop(0, 16, edge_body, 0)
            s = jnp.zeros((16,), jnp.float32)
            for l in range(16):
                col = jnp.full((16,), l, jnp.int32)
                s = s + plsc.load_gather(acc_buf, [lanes, col])
            score_v[pl.ds(g * 16, 16)] = _GAMMA - s
            return carry2

        lax.fori_loop(0, _G16, group_body, 0)
        pltpu.sync_copy(score_v, out_hbm.at[pl.ds(base, _CHUNK)])
        return carry

    lax.fori_loop(0, _NCHUNK, chunk_body, 0)


def kernel(node_emb, edge_index, rel_emb):
    src = edge_index[0].astype(jnp.int32)
    dst = edge_index[1].astype(jnp.int32)
    mesh = plsc.VectorSubcoreMesh(core_axis_name="c", subcore_axis_name="s")
    f = pl.kernel(
        _sc_body,
        out_type=jax.ShapeDtypeStruct((_N_EDGES,), jnp.float32),
        mesh=mesh,
        compiler_params=pltpu.CompilerParams(needs_layout_passes=False),
        scratch_types=[
            pltpu.VMEM((_CHUNK,), jnp.int32),
            pltpu.VMEM((_CHUNK,), jnp.int32),
            pltpu.VMEM((_CHUNK, _D), jnp.float32),
            pltpu.VMEM((_CHUNK, _D), jnp.float32),
            pltpu.VMEM((_CHUNK, _D), jnp.float32),
            pltpu.VMEM((_CHUNK,), jnp.float32),
            pltpu.VMEM((16, 17), jnp.float32),
            pltpu.SemaphoreType.DMA,
        ],
    )
    return f(node_emb, src, dst, rel_emb)


# speedup vs baseline: 5.4417x; 2.5952x over previous
import jax
import jax.numpy as jnp
from jax import lax
from jax.experimental import pallas as pl
from jax.experimental.pallas import tpu as pltpu
from jax.experimental.pallas import tpu_sc as plsc

_GAMMA = 12.0
_N_EDGES = 320000
_D = 128
_NW = 32
_EPW = _N_EDGES // _NW
_CHUNK = 80
_NCHUNK = _EPW // _CHUNK
_NPAIR = (_NCHUNK - 1) // 2
_G16 = _CHUNK // 16


def _sc_body(node_hbm, src_hbm, dst_hbm, rel_hbm, out_hbm,
             src_all, dst_all, out_all,
             head0, tail0, rel0, head1, tail1, rel1,
             acc_buf, sem0, sem1):
    wid = lax.axis_index("s") * 2 + lax.axis_index("c")
    wbase = wid * _EPW
    lanes = lax.iota(jnp.int32, 16)

    pltpu.sync_copy(src_hbm.at[pl.ds(wbase, _EPW)], src_all)
    pltpu.sync_copy(dst_hbm.at[pl.ds(wbase, _EPW)], dst_all)

    def start(ci, head_v, tail_v, rel_v, sem):
        off = ci * _CHUNK
        ch = pltpu.async_copy(node_hbm.at[src_all.at[pl.ds(off, _CHUNK)]],
                              head_v, sem)
        ct = pltpu.async_copy(node_hbm.at[dst_all.at[pl.ds(off, _CHUNK)]],
                              tail_v, sem)
        cr = pltpu.async_copy(rel_hbm.at[pl.ds(wbase + off, _CHUNK)],
                              rel_v, sem)
        return ch, ct, cr

    def wait(copies):
        for c in copies:
            c.wait()

    def compute(ci, head_v, tail_v, rel_v):
        off = ci * _CHUNK

        def group_body(g, carry2):
            def edge_body(e, carry3):
                row = g * 16 + e
                acc = jnp.zeros((16,), jnp.float32)
                for j in range(8):
                    h = head_v[row, pl.ds(j * 16, 16)]
                    r = rel_v[row, pl.ds(j * 16, 16)]
                    t = tail_v[row, pl.ds(j * 16, 16)]
                    acc = acc + jnp.abs(h + r - t)
                acc_buf[e, pl.ds(0, 16)] = acc
                return carry3

            lax.fori_loop(0, 16, edge_body, 0)
            s = jnp.zeros((16,), jnp.float32)
            for l in range(16):
                col = jnp.full((16,), l, jnp.int32)
                s = s + plsc.load_gather(acc_buf, [lanes, col])
            out_all[pl.ds(off + g * 16, 16)] = _GAMMA - s
            return carry2

        lax.fori_loop(0, _G16, group_body, 0)

    c0 = start(0, head0, tail0, rel0, sem0)

    def body(j, carry):
        even = 2 * j
        odd = even + 1
        c_odd = start(odd, head1, tail1, rel1, sem1)
        pltpu.make_async_copy(node_hbm.at[src_all.at[pl.ds(even * _CHUNK, _CHUNK)]], head0, sem0).wait()
        pltpu.make_async_copy(node_hbm.at[dst_all.at[pl.ds(even * _CHUNK, _CHUNK)]], tail0, sem0).wait()
        pltpu.make_async_copy(rel_hbm.at[pl.ds(wbase + even * _CHUNK, _CHUNK)], rel0, sem0).wait()
        compute(even, head0, tail0, rel0)
        c_next = start(even + 2, head0, tail0, rel0, sem0)
        wait(c_odd)
        compute(odd, head1, tail1, rel1)
        return carry

    lax.fori_loop(0, _NPAIR, body, 0)
    pltpu.make_async_copy(node_hbm.at[src_all.at[pl.ds((_NCHUNK - 1) * _CHUNK, _CHUNK)]], head0, sem0).wait()
    pltpu.make_async_copy(node_hbm.at[dst_all.at[pl.ds((_NCHUNK - 1) * _CHUNK, _CHUNK)]], tail0, sem0).wait()
    pltpu.make_async_copy(rel_hbm.at[pl.ds(wbase + (_NCHUNK - 1) * _CHUNK, _CHUNK)], rel0, sem0).wait()
    compute(_NCHUNK - 1, head0, tail0, rel0)

    pltpu.sync_copy(out_all, out_hbm.at[pl.ds(wbase, _EPW)])


def kernel(node_emb, edge_index, rel_emb):
    src = edge_index[0].astype(jnp.int32)
    dst = edge_index[1].astype(jnp.int32)
    mesh = plsc.VectorSubcoreMesh(core_axis_name="c", subcore_axis_name="s")
    f = pl.kernel(
        _sc_body,
        out_type=jax.ShapeDtypeStruct((_N_EDGES,), jnp.float32),
        mesh=mesh,
        compiler_params=pltpu.CompilerParams(needs_layout_passes=False),
        scratch_types=[
            pltpu.VMEM((_EPW,), jnp.int32),
            pltpu.VMEM((_EPW,), jnp.int32),
            pltpu.VMEM((_EPW,), jnp.float32),
            pltpu.VMEM((_CHUNK, _D), jnp.float32),
            pltpu.VMEM((_CHUNK, _D), jnp.float32),
            pltpu.VMEM((_CHUNK, _D), jnp.float32),
            pltpu.VMEM((_CHUNK, _D), jnp.float32),
            pltpu.VMEM((_CHUNK, _D), jnp.float32),
            pltpu.VMEM((_CHUNK, _D), jnp.float32),
            pltpu.VMEM((16, 17), jnp.float32),
            pltpu.SemaphoreType.DMA,
            pltpu.SemaphoreType.DMA,
        ],
    )
    return f(node_emb, src, dst, rel_emb)


# speedup vs baseline: 5.5387x; 1.0178x over previous
import jax
import jax.numpy as jnp
from jax import lax
from jax.experimental import pallas as pl
from jax.experimental.pallas import tpu as pltpu
from jax.experimental.pallas import tpu_sc as plsc

_GAMMA = 12.0
_N_EDGES = 320000
_D = 128
_NW = 32
_EPW = _N_EDGES // _NW
_CHUNK = 80
_NCHUNK = _EPW // _CHUNK
_NPAIR = (_NCHUNK - 1) // 2
_G16 = _CHUNK // 16


def _sc_body(node_hbm, src_hbm, dst_hbm, rel_hbm, out_hbm,
             src_all, dst_all, out_all,
             head0, tail0, rel0, head1, tail1, rel1,
             acc_buf, sem0, sem1):
    wid = lax.axis_index("s") * 2 + lax.axis_index("c")
    wbase = wid * _EPW
    lanes = lax.iota(jnp.int32, 16)

    pltpu.sync_copy(src_hbm.at[pl.ds(wbase, _EPW)], src_all)
    pltpu.sync_copy(dst_hbm.at[pl.ds(wbase, _EPW)], dst_all)

    def start(ci, head_v, tail_v, rel_v, sem):
        off = ci * _CHUNK
        ch = pltpu.async_copy(node_hbm.at[src_all.at[pl.ds(off, _CHUNK)]],
                              head_v, sem)
        ct = pltpu.async_copy(node_hbm.at[dst_all.at[pl.ds(off, _CHUNK)]],
                              tail_v, sem)
        cr = pltpu.async_copy(rel_hbm.at[pl.ds(wbase + off, _CHUNK)],
                              rel_v, sem)
        return ch, ct, cr

    def wait(copies):
        for c in copies:
            c.wait()

    def compute(ci, head_v, tail_v, rel_v):
        off = ci * _CHUNK

        def group_body(g, carry2):
            for e in range(16):
                row = g * 16 + e
                acc = jnp.zeros((16,), jnp.float32)
                for j in range(8):
                    h = head_v[row, pl.ds(j * 16, 16)]
                    r = rel_v[row, pl.ds(j * 16, 16)]
                    t = tail_v[row, pl.ds(j * 16, 16)]
                    acc = acc + jnp.abs(h + r - t)
                acc_buf[e, pl.ds(0, 16)] = acc
            s = jnp.zeros((16,), jnp.float32)
            for l in range(16):
                col = jnp.full((16,), l, jnp.int32)
                s = s + plsc.load_gather(acc_buf, [lanes, col])
            out_all[pl.ds(off + g * 16, 16)] = _GAMMA - s
            return carry2

        lax.fori_loop(0, _G16, group_body, 0)

    c0 = start(0, head0, tail0, rel0, sem0)

    def body(j, carry):
        even = 2 * j
        odd = even + 1
        c_odd = start(odd, head1, tail1, rel1, sem1)
        pltpu.make_async_copy(node_hbm.at[src_all.at[pl.ds(even * _CHUNK, _CHUNK)]], head0, sem0).wait()
        pltpu.make_async_copy(node_hbm.at[dst_all.at[pl.ds(even * _CHUNK, _CHUNK)]], tail0, sem0).wait()
        pltpu.make_async_copy(rel_hbm.at[pl.ds(wbase + even * _CHUNK, _CHUNK)], rel0, sem0).wait()
        compute(even, head0, tail0, rel0)
        c_next = start(even + 2, head0, tail0, rel0, sem0)
        wait(c_odd)
        compute(odd, head1, tail1, rel1)
        return carry

    lax.fori_loop(0, _NPAIR, body, 0)
    pltpu.make_async_copy(node_hbm.at[src_all.at[pl.ds((_NCHUNK - 1) * _CHUNK, _CHUNK)]], head0, sem0).wait()
    pltpu.make_async_copy(node_hbm.at[dst_all.at[pl.ds((_NCHUNK - 1) * _CHUNK, _CHUNK)]], tail0, sem0).wait()
    pltpu.make_async_copy(rel_hbm.at[pl.ds(wbase + (_NCHUNK - 1) * _CHUNK, _CHUNK)], rel0, sem0).wait()
    compute(_NCHUNK - 1, head0, tail0, rel0)

    pltpu.sync_copy(out_all, out_hbm.at[pl.ds(wbase, _EPW)])


def kernel(node_emb, edge_index, rel_emb):
    src = edge_index[0].astype(jnp.int32)
    dst = edge_index[1].astype(jnp.int32)
    mesh = plsc.VectorSubcoreMesh(core_axis_name="c", subcore_axis_name="s")
    f = pl.kernel(
        _sc_body,
        out_type=jax.ShapeDtypeStruct((_N_EDGES,), jnp.float32),
        mesh=mesh,
        compiler_params=pltpu.CompilerParams(needs_layout_passes=False),
        scratch_types=[
            pltpu.VMEM((_EPW,), jnp.int32),
            pltpu.VMEM((_EPW,), jnp.int32),
            pltpu.VMEM((_EPW,), jnp.float32),
            pltpu.VMEM((_CHUNK, _D), jnp.float32),
            pltpu.VMEM((_CHUNK, _D), jnp.float32),
            pltpu.VMEM((_CHUNK, _D), jnp.float32),
            pltpu.VMEM((_CHUNK, _D), jnp.float32),
            pltpu.VMEM((_CHUNK, _D), jnp.float32),
            pltpu.VMEM((_CHUNK, _D), jnp.float32),
            pltpu.VMEM((16, 17), jnp.float32),
            pltpu.SemaphoreType.DMA,
            pltpu.SemaphoreType.DMA,
        ],
    )
    return f(node_emb, src, dst, rel_emb)


# speedup vs baseline: 6.8256x; 1.2323x over previous
import jax
import jax.numpy as jnp
from jax import lax
from jax.experimental import pallas as pl
from jax.experimental.pallas import tpu as pltpu
from jax.experimental.pallas import tpu_sc as plsc

_GAMMA = 12.0
_N_EDGES = 320000
_N_NODES = 10000
_D = 128
_DW = _D // 2
_NW = 32
_EPW = _N_EDGES // _NW
_CHUNK = 80
_NCHUNK = _EPW // _CHUNK
_NPAIR = (_NCHUNK - 1) // 2
_G16 = _CHUNK // 16


def _sc_body(node_hbm, src_hbm, dst_hbm, rel_hbm, out_hbm,
             src_all, dst_all,
             head0, tail0, rel0, score0, head1, tail1, rel1, score1,
             acc_buf, node_sh, sem0, sem1):
    sid = lax.axis_index("s")
    wid = sid * 2 + lax.axis_index("c")
    wbase = wid * _EPW
    lanes = lax.iota(jnp.int32, 16)

    rps = 624
    pltpu.sync_copy(node_hbm.at[pl.ds(sid * rps, rps)],
                    node_sh.at[pl.ds(sid * rps, rps)])

    @pl.when(sid == 0)
    def _copy_tail():
        pltpu.sync_copy(node_hbm.at[pl.ds(16 * rps, _N_NODES - 16 * rps)],
                        node_sh.at[pl.ds(16 * rps, _N_NODES - 16 * rps)])

    pltpu.sync_copy(src_hbm.at[pl.ds(wbase, _EPW)], src_all)
    pltpu.sync_copy(dst_hbm.at[pl.ds(wbase, _EPW)], dst_all)
    plsc.subcore_barrier()

    def start(ci, head_v, tail_v, rel_v, sem):
        off = ci * _CHUNK
        ch = pltpu.async_copy(node_hbm.at[src_all.at[pl.ds(off, _CHUNK)]],
                              head_v, sem)
        ct = pltpu.async_copy(node_hbm.at[dst_all.at[pl.ds(off, _CHUNK)]],
                              tail_v, sem)
        cr = pltpu.async_copy(rel_hbm.at[pl.ds(wbase + off, _CHUNK)],
                              rel_v, sem)
        return ch, ct, cr

    def wait_chunk(ci, head_v, tail_v, rel_v, sem):
        off = ci * _CHUNK
        pltpu.make_async_copy(node_hbm.at[src_all.at[pl.ds(off, _CHUNK)]],
                              head_v, sem).wait()
        pltpu.make_async_copy(node_hbm.at[dst_all.at[pl.ds(off, _CHUNK)]],
                              tail_v, sem).wait()
        pltpu.make_async_copy(rel_hbm.at[pl.ds(wbase + off, _CHUNK)],
                              rel_v, sem).wait()

    def compute(ci, head_v, tail_v, rel_v, score_v):
        off = ci * _CHUNK

        def group_body(g, carry2):
            for e in range(16):
                row = g * 16 + e
                acc = jnp.zeros((16,), jnp.float32)
                for j in range(4):
                    hw = head_v[row, pl.ds(j * 16, 16)]
                    tw = tail_v[row, pl.ds(j * 16, 16)]
                    ha, hb = plsc.unpack(plsc.bitcast(hw, jnp.bfloat16),
                                         format=plsc.PackFormat.INTERLEAVED)
                    ta, tb = plsc.unpack(plsc.bitcast(tw, jnp.bfloat16),
                                         format=plsc.PackFormat.INTERLEAVED)
                    r0 = rel_v[row, pl.ds(j * 32, 16)]
                    r1 = rel_v[row, pl.ds(j * 32 + 16, 16)]
                    acc = acc + jnp.abs(ha + r0 - ta) + jnp.abs(hb + r1 - tb)
                acc_buf[e, pl.ds(0, 16)] = acc
            s = jnp.zeros((16,), jnp.float32)
            for l in range(16):
                col = jnp.full((16,), l, jnp.int32)
                s = s + plsc.load_gather(acc_buf, [lanes, col])
            score_v[pl.ds(g * 16, 16)] = _GAMMA - s
            return carry2

        lax.fori_loop(0, _G16, group_body, 0)
        pltpu.sync_copy(score_v, out_hbm.at[pl.ds(wbase + off, _CHUNK)])

    start(0, head0, tail0, rel0, sem0)

    def body(j, carry):
        even = 2 * j
        start(even + 1, head1, tail1, rel1, sem1)
        wait_chunk(even, head0, tail0, rel0, sem0)
        compute(even, head0, tail0, rel0, score0)
        start(even + 2, head0, tail0, rel0, sem0)
        wait_chunk(even + 1, head1, tail1, rel1, sem1)
        compute(even + 1, head1, tail1, rel1, score1)
        return carry

    lax.fori_loop(0, _NPAIR, body, 0)
    wait_chunk(_NCHUNK - 1, head0, tail0, rel0, sem0)
    compute(_NCHUNK - 1, head0, tail0, rel0, score0)


def kernel(node_emb, edge_index, rel_emb):
    src = edge_index[0].astype(jnp.int32)
    dst = edge_index[1].astype(jnp.int32)
    node_perm = node_emb.reshape(_N_NODES, 4, 2, 16).transpose(0, 1, 3, 2)
    node_bf = node_perm.reshape(_N_NODES, _D).astype(jnp.bfloat16)
    node_packed = jax.lax.bitcast_convert_type(
        node_bf.reshape(_N_NODES, _DW, 2), jnp.float32)

    mesh = plsc.VectorSubcoreMesh(core_axis_name="c", subcore_axis_name="s")
    f = pl.kernel(
        _sc_body,
        out_type=jax.ShapeDtypeStruct((_N_EDGES,), jnp.float32),
        mesh=mesh,
        compiler_params=pltpu.CompilerParams(needs_layout_passes=False,
                                             use_tc_tiling_on_sc=False),
        scratch_types=[
            pltpu.VMEM((_EPW,), jnp.int32),
            pltpu.VMEM((_EPW,), jnp.int32),
            pltpu.VMEM((_CHUNK, _DW), jnp.float32),
            pltpu.VMEM((_CHUNK, _DW), jnp.float32),
            pltpu.VMEM((_CHUNK, _D), jnp.float32),
            pltpu.VMEM((_CHUNK,), jnp.float32),
            pltpu.VMEM((_CHUNK, _DW), jnp.float32),
            pltpu.VMEM((_CHUNK, _DW), jnp.float32),
            pltpu.VMEM((_CHUNK, _D), jnp.float32),
            pltpu.VMEM((_CHUNK,), jnp.float32),
            pltpu.VMEM((16, 17), jnp.float32),
            pltpu.VMEM_SHARED((_N_NODES, _DW), jnp.float32),
            pltpu.SemaphoreType.DMA,
            pltpu.SemaphoreType.DMA,
        ],
    )
    return f(node_packed, src, dst, rel_emb)


# speedup vs baseline: 6.9399x; 1.0167x over previous
import jax
import jax.numpy as jnp
from jax import lax
from jax.experimental import pallas as pl
from jax.experimental.pallas import tpu as pltpu
from jax.experimental.pallas import tpu_sc as plsc

_GAMMA = 12.0
_N_EDGES = 320000
_N_NODES = 10000
_D = 128
_DW = _D // 2
_NW = 32
_EPW = _N_EDGES // _NW
_CHUNK = 80
_NCHUNK = _EPW // _CHUNK
_NPAIR = (_NCHUNK - 1) // 2
_G16 = _CHUNK // 16


def _sc_body(node_hbm, src_hbm, dst_hbm, rel_hbm, out_hbm,
             src_all, dst_all,
             head0, tail0, rel0, score0, head1, tail1, rel1, score1,
             acc_buf, sem0, sem1):
    wid = lax.axis_index("s") * 2 + lax.axis_index("c")
    wbase = wid * _EPW
    lanes = lax.iota(jnp.int32, 16)

    pltpu.sync_copy(src_hbm.at[wid], src_all)
    pltpu.sync_copy(dst_hbm.at[wid], dst_all)

    def start(ci, head_v, tail_v, rel_v, sem):
        off = ci * _CHUNK
        pltpu.async_copy(node_hbm.at[src_all.at[ci]], head_v, sem)
        pltpu.async_copy(node_hbm.at[dst_all.at[ci]], tail_v, sem)
        pltpu.async_copy(rel_hbm.at[pl.ds(wbase + off, _CHUNK)], rel_v, sem)

    def wait_chunk(ci, head_v, tail_v, rel_v, sem):
        off = ci * _CHUNK
        pltpu.make_async_copy(node_hbm.at[src_all.at[ci]], head_v, sem).wait()
        pltpu.make_async_copy(node_hbm.at[dst_all.at[ci]], tail_v, sem).wait()
        pltpu.make_async_copy(rel_hbm.at[pl.ds(wbase + off, _CHUNK)],
                              rel_v, sem).wait()

    def compute(ci, head_v, tail_v, rel_v, score_v):
        off = ci * _CHUNK

        def group_body(g, carry2):
            for e in range(16):
                row = g * 16 + e
                acc = jnp.zeros((16,), jnp.float32)
                for j in range(4):
                    hw = head_v[row, pl.ds(j * 16, 16)]
                    tw = tail_v[row, pl.ds(j * 16, 16)]
                    ha, hb = plsc.unpack(plsc.bitcast(hw, jnp.bfloat16),
                                         format=plsc.PackFormat.INTERLEAVED)
                    ta, tb = plsc.unpack(plsc.bitcast(tw, jnp.bfloat16),
                                         format=plsc.PackFormat.INTERLEAVED)
                    r0 = rel_v[row, pl.ds(j * 32, 16)]
                    r1 = rel_v[row, pl.ds(j * 32 + 16, 16)]
                    acc = acc + jnp.abs(ha + r0 - ta) + jnp.abs(hb + r1 - tb)
                acc_buf[e, pl.ds(0, 16)] = acc
            s = jnp.zeros((16,), jnp.float32)
            for l in range(16):
                col = jnp.full((16,), l, jnp.int32)
                s = s + plsc.load_gather(acc_buf, [lanes, col])
            score_v[pl.ds(g * 16, 16)] = _GAMMA - s
            return carry2

        lax.fori_loop(0, _G16, group_body, 0)
        pltpu.sync_copy(score_v, out_hbm.at[pl.ds(wbase + off, _CHUNK)])

    start(0, head0, tail0, rel0, sem0)

    def body(j, carry):
        even = 2 * j
        start(even + 1, head1, tail1, rel1, sem1)
        wait_chunk(even, head0, tail0, rel0, sem0)
        compute(even, head0, tail0, rel0, score0)
        start(even + 2, head0, tail0, rel0, sem0)
        wait_chunk(even + 1, head1, tail1, rel1, sem1)
        compute(even + 1, head1, tail1, rel1, score1)
        return carry

    lax.fori_loop(0, _NPAIR, body, 0)
    wait_chunk(_NCHUNK - 1, head0, tail0, rel0, sem0)
    compute(_NCHUNK - 1, head0, tail0, rel0, score0)


def kernel(node_emb, edge_index, rel_emb):
    src = edge_index[0].astype(jnp.int32).reshape(_NW, _NCHUNK, _CHUNK)
    dst = edge_index[1].astype(jnp.int32).reshape(_NW, _NCHUNK, _CHUNK)
    node_perm = node_emb.reshape(_N_NODES, 4, 2, 16).transpose(0, 1, 3, 2)
    node_bf = node_perm.reshape(_N_NODES, _D).astype(jnp.bfloat16)
    node_packed = jax.lax.bitcast_convert_type(
        node_bf.reshape(_N_NODES, _DW, 2), jnp.float32)

    mesh = plsc.VectorSubcoreMesh(core_axis_name="c", subcore_axis_name="s")
    f = pl.kernel(
        _sc_body,
        out_type=jax.ShapeDtypeStruct((_N_EDGES,), jnp.float32),
        mesh=mesh,
        compiler_params=pltpu.CompilerParams(needs_layout_passes=False,
                                             use_tc_tiling_on_sc=False),
        scratch_types=[
            pltpu.VMEM((_NCHUNK, _CHUNK), jnp.int32),
            pltpu.VMEM((_NCHUNK, _CHUNK), jnp.int32),
            pltpu.VMEM((_CHUNK, _DW), jnp.float32),
            pltpu.VMEM((_CHUNK, _DW), jnp.float32),
            pltpu.VMEM((_CHUNK, _D), jnp.float32),
            pltpu.VMEM((_CHUNK,), jnp.float32),
            pltpu.VMEM((_CHUNK, _DW), jnp.float32),
            pltpu.VMEM((_CHUNK, _DW), jnp.float32),
            pltpu.VMEM((_CHUNK, _D), jnp.float32),
            pltpu.VMEM((_CHUNK,), jnp.float32),
            pltpu.VMEM((16, 17), jnp.float32),
            pltpu.SemaphoreType.DMA,
            pltpu.SemaphoreType.DMA,
        ],
    )
    return f(node_packed, src, dst, rel_emb)
